# baseline (speedup 1.0000x reference)
import functools

import jax
import jax.numpy as jnp
from jax import lax
from jax.experimental import pallas as pl
from jax.experimental.pallas import tpu as pltpu
from jax.experimental.pallas import tpu_sc as plsc

_NC = 2
_NS = 16
_L = 16
_NW = _NC * _NS
_NPASS = 2


def _body(users_h, items_h, uw_h, iw_h, ub_h, ib_h, out_h,
          uidx, uidxg, ubase, iidx, iidxg, ibase,
          urows, irows, ubv, ibv, outv,
          sem_uw, sem_iw, sem_ub, sem_ib, *, bpw, D):
    wid = lax.axis_index("s") * _NC + lax.axis_index("c")
    base = wid * bpw
    hpw = bpw // _NPASS

    pltpu.sync_copy(users_h.at[pl.ds(base, bpw)], uidx)
    pltpu.sync_copy(items_h.at[pl.ds(base, bpw)], iidx)
    for c in range(bpw // _L):
        s = pl.ds(c * _L, _L)
        um1 = uidx[s] - 1
        im1 = iidx[s] - 1
        uidx[s] = um1
        iidx[s] = im1
        uidxg[s] = lax.shift_right_logical(um1, 1)
        iidxg[s] = lax.shift_right_logical(im1, 1)
        ubase[s] = lax.bitwise_and(um1, 1) * D
        ibase[s] = lax.bitwise_and(im1, 1) * D

    cp_ub = pltpu.async_copy(ub_h.at[uidx], ubv, sem_ub)
    cp_ib = pltpu.async_copy(ib_h.at[iidx], ibv, sem_ib)

    col0 = lax.iota(jnp.int32, _L)

    for p in range(_NPASS):
        half = pl.ds(p * hpw, hpw)
        cp_uw = pltpu.async_copy(uw_h.at[uidxg.at[half]], urows, sem_uw)
        cp_iw = pltpu.async_copy(iw_h.at[iidxg.at[half]], irows, sem_iw)
        if p == 0:
            cp_ub.wait()
            cp_ib.wait()
        cp_uw.wait()
        cp_iw.wait()

        def group(g, carry):
            b0 = p * hpw + g * _L
            row16 = col0 + g * _L
            ucol0 = ubase[pl.ds(b0, _L)]
            icol0 = ibase[pl.ds(b0, _L)]
            acc = ubv[pl.ds(b0, _L)] + ibv[pl.ds(b0, _L)]
            for d in range(D):
                dd = lax.bitwise_and(col0 + d, D - 1)
                uv = plsc.load_gather(urows, [row16, ucol0 + dd])
                iv = plsc.load_gather(irows, [row16, icol0 + dd])
                acc = acc + uv * iv
            outv[pl.ds(b0, _L)] = 5.0 / (1.0 + jnp.exp(-acc))
            return carry

        lax.fori_loop(0, hpw // _L, group, 0)

    pltpu.sync_copy(outv, out_h.at[pl.ds(base, bpw)])


@jax.jit
def kernel(users, items, u_weight, i_weight, u_bias, i_bias):
    B = users.shape[0]
    N, D = u_weight.shape
    fold = 128 // D
    bpw = B // _NW
    hpw = bpw // _NPASS
    mesh = plsc.VectorSubcoreMesh(core_axis_name="c", subcore_axis_name="s")
    f = pl.kernel(
        functools.partial(_body, bpw=bpw, D=D),
        out_type=jax.ShapeDtypeStruct((B,), jnp.float32),
        mesh=mesh,
        compiler_params=pltpu.CompilerParams(needs_layout_passes=False),
        scratch_types=[
            pltpu.VMEM((bpw,), jnp.int32),
            pltpu.VMEM((bpw,), jnp.int32),
            pltpu.VMEM((bpw,), jnp.int32),
            pltpu.VMEM((bpw,), jnp.int32),
            pltpu.VMEM((bpw,), jnp.int32),
            pltpu.VMEM((bpw,), jnp.int32),
            pltpu.VMEM((hpw, D * fold), jnp.float32),
            pltpu.VMEM((hpw, D * fold), jnp.float32),
            pltpu.VMEM((bpw,), jnp.float32),
            pltpu.VMEM((bpw,), jnp.float32),
            pltpu.VMEM((bpw,), jnp.float32),
            pltpu.SemaphoreType.DMA,
            pltpu.SemaphoreType.DMA,
            pltpu.SemaphoreType.DMA,
            pltpu.SemaphoreType.DMA,
        ],
    )
    return f(users, items,
             u_weight.reshape(N // fold, D * fold),
             i_weight.reshape(N // fold, D * fold),
             u_bias.reshape(-1), i_bias.reshape(-1))


# speedup vs baseline: 2.2214x; 2.2214x over previous
import functools

import jax
import jax.numpy as jnp
from jax import lax
from jax.experimental import pallas as pl
from jax.experimental.pallas import tpu as pltpu
from jax.experimental.pallas import tpu_sc as plsc

_NC = 2
_NS = 16
_L = 16
_NW = _NC * _NS
_G = 8
_CH = 32


def _body(users_h, items_h, uw_h, iw_h, ub_h, ib_h, out_h,
          uidx, iidx, usub, isub, u3, i3, ubv, ibv, outv,
          sem_uw, sem_iw, sem_ub, sem_ib, *, bpw, D):
    wid = lax.axis_index("s") * _NC + lax.axis_index("c")
    base = wid * bpw

    pltpu.sync_copy(users_h.at[pl.ds(base, bpw)], uidx)
    pltpu.sync_copy(items_h.at[pl.ds(base, bpw)], iidx)
    for c in range(bpw // _L):
        s = pl.ds(c * _L, _L)
        um1 = uidx[s] - 1
        im1 = iidx[s] - 1
        uidx[s] = um1
        iidx[s] = im1
        usub[s] = lax.bitwise_and(um1, _G - 1)
        isub[s] = lax.bitwise_and(im1, _G - 1)
    cp_ub = pltpu.async_copy(ub_h.at[uidx], ubv, sem_ub)
    cp_ib = pltpu.async_copy(ib_h.at[iidx], ibv, sem_ib)
    cp_ub.wait()
    cp_ib.wait()

    col0 = lax.iota(jnp.int32, _L)

    def chunk(k, carry):
        c0 = k * _CH

        for q in range(_CH // _L):
            ug16 = lax.shift_right_logical(
                uidx[pl.ds(c0 + q * _L, _L)], 3)
            ig16 = lax.shift_right_logical(
                iidx[pl.ds(c0 + q * _L, _L)], 3)
            for l in range(_L):
                jj = q * _L + l
                pltpu.async_copy(uw_h.at[ug16[l]], u3.at[jj], sem_uw)
                pltpu.async_copy(iw_h.at[ig16[l]], i3.at[jj], sem_iw)

        for q in range(_CH // _L):
            ug16 = lax.shift_right_logical(
                uidx[pl.ds(c0 + q * _L, _L)], 3)
            ig16 = lax.shift_right_logical(
                iidx[pl.ds(c0 + q * _L, _L)], 3)
            for l in range(_L):
                jj = q * _L + l
                pltpu.make_async_copy(uw_h.at[ug16[l]], u3.at[jj],
                                      sem_uw).wait()
                pltpu.make_async_copy(iw_h.at[ig16[l]], i3.at[jj],
                                      sem_iw).wait()

        for g in range(_CH // _L):
            b0 = c0 + g * _L
            row16 = col0 + g * _L
            us16 = usub[pl.ds(b0, _L)]
            is16 = isub[pl.ds(b0, _L)]
            acc = ubv[pl.ds(b0, _L)] + ibv[pl.ds(b0, _L)]
            for d in range(D):
                dd = lax.bitwise_and(col0 + d, D - 1)
                uv = plsc.load_gather(u3, [row16, us16, dd])
                iv = plsc.load_gather(i3, [row16, is16, dd])
                acc = acc + uv * iv
            outv[pl.ds(b0, _L)] = 5.0 / (1.0 + jnp.exp(-acc))
        return carry

    lax.fori_loop(0, bpw // _CH, chunk, 0)

    pltpu.sync_copy(outv, out_h.at[pl.ds(base, bpw)])


@jax.jit
def kernel(users, items, u_weight, i_weight, u_bias, i_bias):
    B = users.shape[0]
    N, D = u_weight.shape
    bpw = B // _NW
    mesh = plsc.VectorSubcoreMesh(core_axis_name="c", subcore_axis_name="s")
    f = pl.kernel(
        functools.partial(_body, bpw=bpw, D=D),
        out_type=jax.ShapeDtypeStruct((B,), jnp.float32),
        mesh=mesh,
        compiler_params=pltpu.CompilerParams(needs_layout_passes=False),
        scratch_types=[
            pltpu.VMEM((bpw,), jnp.int32),
            pltpu.VMEM((bpw,), jnp.int32),
            pltpu.VMEM((bpw,), jnp.int32),
            pltpu.VMEM((bpw,), jnp.int32),
            pltpu.VMEM((_CH, _G, D), jnp.float32),
            pltpu.VMEM((_CH, _G, D), jnp.float32),
            pltpu.VMEM((bpw,), jnp.float32),
            pltpu.VMEM((bpw,), jnp.float32),
            pltpu.VMEM((bpw,), jnp.float32),
            pltpu.SemaphoreType.DMA,
            pltpu.SemaphoreType.DMA,
            pltpu.SemaphoreType.DMA,
            pltpu.SemaphoreType.DMA,
        ],
    )
    return f(users, items,
             u_weight.reshape(N // _G, _G, D),
             i_weight.reshape(N // _G, _G, D),
             u_bias.reshape(-1), i_bias.reshape(-1))
